# baseline (speedup 1.0000x reference)
import jax
import jax.numpy as jnp
import numpy as np
from jax.experimental import pallas as pl
from jax.experimental.pallas import tpu as pltpu

EMBED = 512


def _make_pe(d_model, max_len):
    position = np.arange(max_len, dtype=np.float32)[:, None]
    div_term = np.exp(
        np.arange(0, d_model, 2, dtype=np.float32) * (-np.log(10000.0) / d_model)
    )
    pe = np.zeros((max_len, d_model), dtype=np.float32)
    pe[:, 0::2] = np.sin(position * div_term)
    pe[:, 1::2] = np.cos(position * div_term)
    return pe


def _postnet_kernel(
    enc_ref,
    ap_ref,
    tp_ref,
    pitch_ref,
    beats_ref,
    wp_ref,
    bp_ref,
    wt_ref,
    bpos_ref,
    emb_ref,
    pe_ref,
    out_ref,
    idx_scr,
    pew_scr,
):
    b = pl.program_id(0)
    B, T_frame = ap_ref.shape
    T_text = tp_ref.shape[1]
    D = wt_ref.shape[0]

    @pl.when(b == 0)
    def _prologue():
        pew_scr[...] = (
            jnp.dot(pe_ref[...], wt_ref[...], preferred_element_type=jnp.float32)
            + bpos_ref[...]
            + bp_ref[...]
            + emb_ref[0:1, :]
        )

        a = ap_ref[...]
        iota = jax.lax.broadcasted_iota(jnp.int32, (B, T_frame), 1)
        sentinel = jnp.int32(2 * T_frame)
        jp = jnp.zeros((B, 1), jnp.int32)
        acc = jnp.zeros((B, T_frame), jnp.int32)
        for p in range(T_text - 1):
            tp_col = tp_ref[:, p : p + 1]
            mask = (a != tp_col) & (iota > jp)
            cand = jnp.where(mask, iota, sentinel)
            jn = jnp.min(cand, axis=1, keepdims=True)
            acc = acc + (iota >= jn).astype(jnp.int32)
            jp = jn
        idx_scr[...] = acc.astype(jnp.float32)

    bhot = (
        jax.lax.broadcasted_iota(jnp.int32, (B, 1), 0) == b
    ).astype(jnp.float32)
    idx_row = jnp.dot(
        bhot.T, idx_scr[...], preferred_element_type=jnp.float32
    )
    pitch_col = jnp.dot(pitch_ref[...], bhot, preferred_element_type=jnp.float32)
    beats_col = jnp.dot(beats_ref[...], bhot, preferred_element_type=jnp.float32)

    enc = enc_ref[0]
    g = enc + jnp.dot(
        enc.astype(jnp.bfloat16), wt_ref[...], preferred_element_type=jnp.float32
    )
    oh_t = (
        jax.lax.broadcasted_iota(jnp.int32, (T_text, T_frame), 0)
        == idx_row.astype(jnp.int32)
    ).astype(jnp.bfloat16)
    gathered = jax.lax.dot_general(
        oh_t,
        g.astype(jnp.bfloat16),
        dimension_numbers=(((0,), (0,)), ((), ())),
        preferred_element_type=jnp.float32,
    )

    demb = emb_ref[1:2, :] - emb_ref[0:1, :]
    out_ref[0] = (
        gathered
        + pitch_col * wp_ref[...]
        + beats_col * demb
        + pew_scr[...]
    )


@jax.jit
def kernel(
    encoder_out,
    align_phone,
    text_phone,
    pitch,
    beats,
    fc_pitch_w,
    fc_pitch_b,
    fc_pos_w,
    fc_pos_b,
    emb_beats,
):
    B, T_text, D = encoder_out.shape
    T_frame = align_phone.shape[1]

    ap = align_phone.astype(jnp.int32)
    tp = text_phone.astype(jnp.int32)
    pitch_t = jnp.squeeze(pitch, -1).T
    beats_t = jnp.squeeze(beats, -1).astype(jnp.float32).T
    wp = fc_pitch_w.reshape(1, D)
    bp = fc_pitch_b.reshape(1, D)
    wt = fc_pos_w.T.astype(jnp.bfloat16)
    bpos = fc_pos_b.reshape(1, D)
    pe = jnp.asarray(_make_pe(D, T_frame)).astype(jnp.bfloat16)

    grid = (B,)
    out = pl.pallas_call(
        _postnet_kernel,
        grid=grid,
        in_specs=[
            pl.BlockSpec((1, T_text, D), lambda b: (b, 0, 0)),
            pl.BlockSpec((B, T_frame), lambda b: (0, 0)),
            pl.BlockSpec((B, T_text), lambda b: (0, 0)),
            pl.BlockSpec((T_frame, B), lambda b: (0, 0)),
            pl.BlockSpec((T_frame, B), lambda b: (0, 0)),
            pl.BlockSpec((1, D), lambda b: (0, 0)),
            pl.BlockSpec((1, D), lambda b: (0, 0)),
            pl.BlockSpec((D, D), lambda b: (0, 0)),
            pl.BlockSpec((1, D), lambda b: (0, 0)),
            pl.BlockSpec((2, D), lambda b: (0, 0)),
            pl.BlockSpec((T_frame, D), lambda b: (0, 0)),
        ],
        out_specs=pl.BlockSpec((1, T_frame, D), lambda b: (b, 0, 0)),
        out_shape=jax.ShapeDtypeStruct((B, T_frame, D), jnp.float32),
        scratch_shapes=[
            pltpu.VMEM((B, T_frame), jnp.float32),
            pltpu.VMEM((T_frame, D), jnp.float32),
        ],
        compiler_params=pltpu.CompilerParams(
            dimension_semantics=("arbitrary",),
        ),
    )(
        encoder_out,
        ap,
        tp,
        pitch_t,
        beats_t,
        wp,
        bp,
        wt,
        bpos,
        emb_beats,
        pe,
    )
    return out
